# baseline (speedup 1.0000x reference)
import functools

import jax
import jax.numpy as jnp
from jax.experimental import pallas as pl
from jax.experimental.pallas import tpu as pltpu

_NUM_CLASSES = 9
_IN_CH = 103
_P = 5
_PP = _P * _P
_CPAD = 128
_KF = _PP * _CPAD
_BN_EPS = 1e-5


def _round_up(v, m):
    return (v + m - 1) // m * m


def _tap_map():
    taps = {}
    for oi in range(3):
        for oj in range(3):
            for ki in range(3):
                for kj in range(3):
                    taps[(oi * 3 + oj, (oi + ki) * _P + (oj + kj))] = ki * 3 + kj
    return taps


_TAPS = _tap_map()


def _feat_kernel(x_ref, pa_ref, w1b_ref, ps_ref,
                 feat_ref, s1_ref, s2_ref, xs_ref, w_ref):
    tn = feat_ref.shape[1]
    j = pl.program_id(1)

    @pl.when(j == 0)
    def _build_w():
        w_ref[...] = jnp.zeros_like(w_ref)
        for t in range(9):
            blk = pa_ref[t * 64:(t + 1) * 64, :].astype(jnp.bfloat16)
            for (o, p), tt in _TAPS.items():
                if tt == t:
                    w_ref[o * 64:(o + 1) * 64,
                          p * _CPAD:p * _CPAD + _IN_CH] = blk
        zpad = jnp.zeros((_CPAD - _IN_CH, tn), jnp.bfloat16)
        for p in range(_PP):
            xs_ref[p * _CPAD + _IN_CH:(p + 1) * _CPAD, :] = zpad

    for p in range(_PP):
        xs_ref[p * _CPAD:p * _CPAD + _IN_CH, :] = x_ref[p].astype(jnp.bfloat16)
    xs = xs_ref[...]

    y1a = jnp.dot(w_ref[...], xs,
                  preferred_element_type=jnp.float32)
    m1a = y1a[0:64]
    for o in range(1, 9):
        m1a = jnp.maximum(m1a, y1a[o * 64:(o + 1) * 64])

    w1b = w1b_ref[...].astype(jnp.bfloat16)
    m1b = None
    for p in range(_PP):
        z = jnp.dot(w1b, xs_ref[p * _CPAD:p * _CPAD + _IN_CH, :],
                    preferred_element_type=jnp.float32)
        m1b = z if m1b is None else jnp.maximum(m1b, z)

    feat = jnp.concatenate([m1a, m1b], axis=0) + ps_ref[:, 0:1]
    feat_ref[...] = feat.astype(feat_ref.dtype)

    ones = jnp.ones((tn, 128), jnp.float32)
    s1_ref[...] = jnp.dot(feat, ones,
                          preferred_element_type=jnp.float32)[None]
    s2_ref[...] = jnp.dot(feat * feat, ones,
                          preferred_element_type=jnp.float32)[None]


def _head_kernel(n_real, n_fake, feat_ref, s1_ref, s2_ref, ps_ref,
                 w2a_ref, w2b_ref, wfc_ref, out_ref):
    bias = ps_ref[:, 0:1]
    s1 = jnp.sum(s1_ref[...], axis=0)[:, 0:1] - n_fake * bias
    s2 = jnp.sum(s2_ref[...], axis=0)[:, 0:1] - n_fake * bias * bias
    mean = s1 / n_real
    var = jnp.maximum(s2 / n_real - mean * mean, 0.0)
    inv = jax.lax.rsqrt(var + _BN_EPS)
    scale = ps_ref[:, 1:2] * inv
    shift = ps_ref[:, 2:3] - mean * scale

    feat = feat_ref[...].astype(jnp.float32)
    h = jnp.maximum(feat * scale + shift, 0.0)
    h = jnp.dot(w2a_ref[...].astype(jnp.bfloat16), h.astype(jnp.bfloat16),
                preferred_element_type=jnp.float32) + ps_ref[:, 3:4]
    h = jnp.maximum(h, 0.0)
    h = jnp.dot(w2b_ref[...].astype(jnp.bfloat16), h.astype(jnp.bfloat16),
                preferred_element_type=jnp.float32) + ps_ref[:, 4:5]
    res = feat + h
    out = jnp.dot(wfc_ref[...].astype(jnp.bfloat16), res.astype(jnp.bfloat16),
                  preferred_element_type=jnp.float32) + ps_ref[0:_NUM_CLASSES, 5:6]
    out_bf = out.astype(jnp.bfloat16)
    out_ref[...] = out_bf.astype(jnp.float32)


@functools.partial(jax.jit, static_argnames=("tile_n",))
def _forward(x, w1a, b1a, w1b, b1b, gamma, beta, w2a, b2a, w2b, b2b,
             wfc, bfc, tile_n=256):
    n = x.shape[0]
    tn = min(tile_n, _round_up(n, 128))
    n_pad = _round_up(n, tn)
    grid_n = n_pad // tn
    ncores = 2 if grid_n % 2 == 0 else 1
    g2 = grid_n // ncores

    xt = x.transpose(2, 3, 1, 0).reshape(_PP, _IN_CH, n)
    if n_pad != n:
        xt = jnp.pad(xt, ((0, 0), (0, 0), (0, n_pad - n)))

    pack_a = w1a.transpose(2, 3, 0, 1).reshape(576, _IN_CH)

    w1b_v = w1b.reshape(64, _IN_CH)
    w2a_v = w2a.reshape(128, 128)
    w2b_v = w2b.reshape(128, 128)
    bias = jnp.concatenate([b1a, b1b])
    bfc_p = jnp.pad(bfc, (0, 128 - _NUM_CLASSES))
    zcol = jnp.zeros((128,), jnp.float32)
    pack_s = jnp.stack([bias, gamma, beta, b2a, b2b, bfc_p, zcol, zcol],
                       axis=1)

    cparams1 = pltpu.CompilerParams(
        dimension_semantics=("parallel", "arbitrary"),
        vmem_limit_bytes=100 << 20,
    )
    cparams2 = pltpu.CompilerParams(
        dimension_semantics=("parallel",),
        vmem_limit_bytes=100 << 20,
    )

    feat, s1, s2 = pl.pallas_call(
        _feat_kernel,
        out_shape=(
            jax.ShapeDtypeStruct((128, n_pad), jnp.bfloat16),
            jax.ShapeDtypeStruct((grid_n, 128, 128), jnp.float32),
            jax.ShapeDtypeStruct((grid_n, 128, 128), jnp.float32),
        ),
        grid=(ncores, g2),
        in_specs=[
            pl.BlockSpec((_PP, _IN_CH, tn), lambda c, j: (0, 0, c * g2 + j)),
            pl.BlockSpec((576, _IN_CH), lambda c, j: (0, 0)),
            pl.BlockSpec((64, _IN_CH), lambda c, j: (0, 0)),
            pl.BlockSpec((128, 8), lambda c, j: (0, 0)),
        ],
        out_specs=(
            pl.BlockSpec((128, tn), lambda c, j: (0, c * g2 + j)),
            pl.BlockSpec((1, 128, 128), lambda c, j: (c * g2 + j, 0, 0)),
            pl.BlockSpec((1, 128, 128), lambda c, j: (c * g2 + j, 0, 0)),
        ),
        scratch_shapes=[
            pltpu.VMEM((_KF, tn), jnp.bfloat16),
            pltpu.VMEM((576, _KF), jnp.bfloat16),
        ],
        compiler_params=cparams1,
    )(xt, pack_a, w1b_v, pack_s)

    tn2 = n_pad // ncores if ncores == 2 else tn
    grid_h = n_pad // tn2
    kern = functools.partial(_head_kernel, float(n), float(n_pad - n))
    out_t = pl.pallas_call(
        kern,
        out_shape=jax.ShapeDtypeStruct((_NUM_CLASSES, n_pad), jnp.float32),
        grid=(grid_h,),
        in_specs=[
            pl.BlockSpec((128, tn2), lambda i: (0, i)),
            pl.BlockSpec((grid_n, 128, 128), lambda i: (0, 0, 0)),
            pl.BlockSpec((grid_n, 128, 128), lambda i: (0, 0, 0)),
            pl.BlockSpec((128, 8), lambda i: (0, 0)),
            pl.BlockSpec((128, 128), lambda i: (0, 0)),
            pl.BlockSpec((128, 128), lambda i: (0, 0)),
            pl.BlockSpec((_NUM_CLASSES, 128), lambda i: (0, 0)),
        ],
        out_specs=pl.BlockSpec((_NUM_CLASSES, tn2), lambda i: (0, i)),
        compiler_params=cparams2,
    )(feat, s1, s2, pack_s, w2a_v, w2b_v, wfc)

    return out_t[:, :n].T


def kernel(x, w1a, b1a, w1b, b1b, gamma, beta, w2a, b2a, w2b, b2b, wfc, bfc):
    return _forward(x, w1a, b1a, w1b, b1b, gamma, beta, w2a, b2a, w2b, b2b,
                    wfc, bfc, tile_n=256)


# speedup vs baseline: 1.1383x; 1.1383x over previous
import functools

import jax
import jax.numpy as jnp
from jax.experimental import pallas as pl
from jax.experimental.pallas import tpu as pltpu

_NUM_CLASSES = 9
_IN_CH = 103
_P = 5
_PP = _P * _P
_CPAD = 128
_KF = _PP * _CPAD
_BN_EPS = 1e-5


def _round_up(v, m):
    return (v + m - 1) // m * m


def _tap_map():
    taps = {}
    for oi in range(3):
        for oj in range(3):
            for ki in range(3):
                for kj in range(3):
                    taps[(oi * 3 + oj, (oi + ki) * _P + (oj + kj))] = ki * 3 + kj
    return taps


_TAPS = _tap_map()


def _feat_kernel(x_ref, pa_ref, w1b_ref, ps_ref,
                 feat_ref, s1_ref, s2_ref, xs_ref, w_ref, wb2_ref):
    tn = feat_ref.shape[1]
    j = pl.program_id(1)

    @pl.when(j == 0)
    def _build_w():
        w_ref[...] = jnp.zeros_like(w_ref)
        for t in range(9):
            blk = pa_ref[t * 64:(t + 1) * 64, :].astype(jnp.bfloat16)
            for (o, p), tt in _TAPS.items():
                if tt == t:
                    w_ref[o * 64:(o + 1) * 64,
                          p * _CPAD:p * _CPAD + _IN_CH] = blk
        zpad = jnp.zeros((_CPAD - _IN_CH, tn), jnp.bfloat16)
        for p in range(_PP):
            xs_ref[p * _CPAD + _IN_CH:(p + 1) * _CPAD, :] = zpad
        wb2_ref[...] = jnp.zeros_like(wb2_ref)
        wb = w1b_ref[...].astype(jnp.bfloat16)
        wb2_ref[0:64, 0:_IN_CH] = wb
        wb2_ref[64:128, _CPAD:_CPAD + _IN_CH] = wb

    for p in range(_PP):
        xs_ref[p * _CPAD:p * _CPAD + _IN_CH, :] = x_ref[p].astype(jnp.bfloat16)
    xs = xs_ref[...]

    y1a = jnp.dot(w_ref[...], xs,
                  preferred_element_type=jnp.float32)
    m1a = y1a[0:64]
    for o in range(1, 9):
        m1a = jnp.maximum(m1a, y1a[o * 64:(o + 1) * 64])

    wb2 = wb2_ref[...]
    mp = None
    for q in range(_PP // 2):
        z = jnp.dot(wb2, xs_ref[q * 2 * _CPAD:(q * 2 + 2) * _CPAD, :],
                    preferred_element_type=jnp.float32)
        mp = z if mp is None else jnp.maximum(mp, z)
    z24 = jnp.dot(wb2[0:64, 0:_CPAD],
                  xs_ref[(_PP - 1) * _CPAD:_PP * _CPAD, :],
                  preferred_element_type=jnp.float32)
    m1b = jnp.maximum(jnp.maximum(mp[0:64], mp[64:128]), z24)

    feat = jnp.concatenate([m1a, m1b], axis=0) + ps_ref[:, 0:1]
    feat_ref[...] = feat.astype(feat_ref.dtype)

    ones = jnp.ones((tn, 128), jnp.float32)
    s1_ref[...] = jnp.dot(feat, ones,
                          preferred_element_type=jnp.float32)[None]
    s2_ref[...] = jnp.dot(feat * feat, ones,
                          preferred_element_type=jnp.float32)[None]


def _head_kernel(n_real, n_fake, feat_ref, s1_ref, s2_ref, ps_ref,
                 w2a_ref, w2b_ref, wfc_ref, out_ref):
    bias = ps_ref[:, 0:1]
    s1 = jnp.sum(s1_ref[...], axis=0)[:, 0:1] - n_fake * bias
    s2 = jnp.sum(s2_ref[...], axis=0)[:, 0:1] - n_fake * bias * bias
    mean = s1 / n_real
    var = jnp.maximum(s2 / n_real - mean * mean, 0.0)
    inv = jax.lax.rsqrt(var + _BN_EPS)
    scale = ps_ref[:, 1:2] * inv
    shift = ps_ref[:, 2:3] - mean * scale

    feat = feat_ref[...].astype(jnp.float32)
    h = jnp.maximum(feat * scale + shift, 0.0)
    h = jnp.dot(w2a_ref[...].astype(jnp.bfloat16), h.astype(jnp.bfloat16),
                preferred_element_type=jnp.float32) + ps_ref[:, 3:4]
    h = jnp.maximum(h, 0.0)
    h = jnp.dot(w2b_ref[...].astype(jnp.bfloat16), h.astype(jnp.bfloat16),
                preferred_element_type=jnp.float32) + ps_ref[:, 4:5]
    res = feat + h
    out = jnp.dot(wfc_ref[...].astype(jnp.bfloat16), res.astype(jnp.bfloat16),
                  preferred_element_type=jnp.float32) + ps_ref[0:_NUM_CLASSES, 5:6]
    out_bf = out.astype(jnp.bfloat16)
    out_ref[...] = out_bf.astype(jnp.float32)


@functools.partial(jax.jit, static_argnames=("tile_n",))
def _forward(x, w1a, b1a, w1b, b1b, gamma, beta, w2a, b2a, w2b, b2b,
             wfc, bfc, tile_n=512):
    n = x.shape[0]
    tn = min(tile_n, _round_up(n, 128))
    n_pad = _round_up(n, tn)
    grid_n = n_pad // tn
    ncores = 2 if grid_n % 2 == 0 else 1
    g2 = grid_n // ncores

    xt = x.transpose(2, 3, 1, 0).reshape(_PP, _IN_CH, n)
    if n_pad != n:
        xt = jnp.pad(xt, ((0, 0), (0, 0), (0, n_pad - n)))

    pack_a = w1a.transpose(2, 3, 0, 1).reshape(576, _IN_CH)

    w1b_v = w1b.reshape(64, _IN_CH)
    w2a_v = w2a.reshape(128, 128)
    w2b_v = w2b.reshape(128, 128)
    bias = jnp.concatenate([b1a, b1b])
    bfc_p = jnp.pad(bfc, (0, 128 - _NUM_CLASSES))
    zcol = jnp.zeros((128,), jnp.float32)
    pack_s = jnp.stack([bias, gamma, beta, b2a, b2b, bfc_p, zcol, zcol],
                       axis=1)

    cparams1 = pltpu.CompilerParams(
        dimension_semantics=("parallel", "arbitrary"),
        vmem_limit_bytes=100 << 20,
    )
    cparams2 = pltpu.CompilerParams(
        dimension_semantics=("parallel",),
        vmem_limit_bytes=100 << 20,
    )

    feat, s1, s2 = pl.pallas_call(
        _feat_kernel,
        out_shape=(
            jax.ShapeDtypeStruct((128, n_pad), jnp.bfloat16),
            jax.ShapeDtypeStruct((grid_n, 128, 128), jnp.float32),
            jax.ShapeDtypeStruct((grid_n, 128, 128), jnp.float32),
        ),
        grid=(ncores, g2),
        in_specs=[
            pl.BlockSpec((_PP, _IN_CH, tn), lambda c, j: (0, 0, c * g2 + j)),
            pl.BlockSpec((576, _IN_CH), lambda c, j: (0, 0)),
            pl.BlockSpec((64, _IN_CH), lambda c, j: (0, 0)),
            pl.BlockSpec((128, 8), lambda c, j: (0, 0)),
        ],
        out_specs=(
            pl.BlockSpec((128, tn), lambda c, j: (0, c * g2 + j)),
            pl.BlockSpec((1, 128, 128), lambda c, j: (c * g2 + j, 0, 0)),
            pl.BlockSpec((1, 128, 128), lambda c, j: (c * g2 + j, 0, 0)),
        ),
        scratch_shapes=[
            pltpu.VMEM((_KF, tn), jnp.bfloat16),
            pltpu.VMEM((576, _KF), jnp.bfloat16),
            pltpu.VMEM((128, 2 * _CPAD), jnp.bfloat16),
        ],
        compiler_params=cparams1,
    )(xt, pack_a, w1b_v, pack_s)

    tn2 = n_pad // ncores if ncores == 2 else tn
    grid_h = n_pad // tn2
    kern = functools.partial(_head_kernel, float(n), float(n_pad - n))
    out_t = pl.pallas_call(
        kern,
        out_shape=jax.ShapeDtypeStruct((_NUM_CLASSES, n_pad), jnp.float32),
        grid=(grid_h,),
        in_specs=[
            pl.BlockSpec((128, tn2), lambda i: (0, i)),
            pl.BlockSpec((grid_n, 128, 128), lambda i: (0, 0, 0)),
            pl.BlockSpec((grid_n, 128, 128), lambda i: (0, 0, 0)),
            pl.BlockSpec((128, 8), lambda i: (0, 0)),
            pl.BlockSpec((128, 128), lambda i: (0, 0)),
            pl.BlockSpec((128, 128), lambda i: (0, 0)),
            pl.BlockSpec((_NUM_CLASSES, 128), lambda i: (0, 0)),
        ],
        out_specs=pl.BlockSpec((_NUM_CLASSES, tn2), lambda i: (0, i)),
        compiler_params=cparams2,
    )(feat, s1, s2, pack_s, w2a_v, w2b_v, wfc)

    return out_t[:, :n].T


def kernel(x, w1a, b1a, w1b, b1b, gamma, beta, w2a, b2a, w2b, b2b, wfc, bfc):
    return _forward(x, w1a, b1a, w1b, b1b, gamma, beta, w2a, b2a, w2b, b2b,
                    wfc, bfc, tile_n=512)
